# Initial kernel scaffold; baseline (speedup 1.0000x reference)
import jax
import jax.numpy as jnp
from jax.experimental import pallas as pl


def kernel(node_attr, pos, edge_attr, spec, Wp, bp, g0, b0, We1, be1, We2, be2, att, Wm, bm, Wv, bv, Ws, bs, gam, bet, Wf1, bf1, Wf2, bf2, edge_index, batch, source, sink):
    raise NotImplementedError("write your pallas kernel here")


# speedup vs baseline: 1.0273x; 1.0273x over previous
import functools
import jax
import jax.numpy as jnp
from jax.experimental import pallas as pl

N_NODES = 10000
N_EDGES = 320000
D = 128
L_LAYERS = 2
K_HOPS = 2

EBLK = 512


def _bn(x, gamma, beta, eps=1e-5):
    mu = jnp.mean(x, axis=0, keepdims=True)
    var = jnp.var(x, axis=0, keepdims=True)
    return gamma * (x - mu) / jnp.sqrt(var + eps) + beta


def _enc_body(ag_ref, bg_ref, ea_ref, we1e_ref, be1_ref, we2_ref, be2_ref, ae_ref,
              ne_ref, nea_ref):
    eh = ag_ref[...] + bg_ref[...] + ea_ref[...] @ we1e_ref[...] + be1_ref[...]
    eh = jnp.maximum(eh, 0.0)
    ne = eh @ we2_ref[...] + be2_ref[...]
    ne_ref[...] = ne
    nea_ref[...] = ne @ ae_ref[...]


def _encoder(ag, bg, ea, we1e, be1, we2, be2, ae):
    ein = ea.shape[1]
    grid = (N_EDGES // EBLK,)
    return pl.pallas_call(
        _enc_body,
        grid=grid,
        in_specs=[
            pl.BlockSpec((EBLK, D), lambda i: (i, 0)),
            pl.BlockSpec((EBLK, D), lambda i: (i, 0)),
            pl.BlockSpec((EBLK, ein), lambda i: (i, 0)),
            pl.BlockSpec((ein, D), lambda i: (0, 0)),
            pl.BlockSpec((1, D), lambda i: (0, 0)),
            pl.BlockSpec((D, ein), lambda i: (0, 0)),
            pl.BlockSpec((1, ein), lambda i: (0, 0)),
            pl.BlockSpec((ein,), lambda i: (0,)),
        ],
        out_specs=[
            pl.BlockSpec((EBLK, ein), lambda i: (i, 0)),
            pl.BlockSpec((EBLK,), lambda i: (i,)),
        ],
        out_shape=[
            jax.ShapeDtypeStruct((N_EDGES, ein), jnp.float32),
            jax.ShapeDtypeStruct((N_EDGES,), jnp.float32),
        ],
    )(ag, bg, ea, we1e, be1[None, :], we2, be2[None, :], ae)


def _hop_body(g1s_ref, g2d_ref, nea_ref, mg_ref, msg_ref, w_ref):
    logit = g1s_ref[...] + g2d_ref[...] + nea_ref[...]
    logit = jnp.where(logit >= 0.0, logit, 0.2 * logit)
    w = jnp.exp(logit)
    w_ref[...] = w
    msg_ref[...] = mg_ref[...] * w[:, None]


def _hop_stage(g1s, g2d, nea, mg):
    grid = (N_EDGES // EBLK,)
    return pl.pallas_call(
        _hop_body,
        grid=grid,
        in_specs=[
            pl.BlockSpec((EBLK,), lambda i: (i,)),
            pl.BlockSpec((EBLK,), lambda i: (i,)),
            pl.BlockSpec((EBLK,), lambda i: (i,)),
            pl.BlockSpec((EBLK, D), lambda i: (i, 0)),
        ],
        out_specs=[
            pl.BlockSpec((EBLK, D), lambda i: (i, 0)),
            pl.BlockSpec((EBLK,), lambda i: (i,)),
        ],
        out_shape=[
            jax.ShapeDtypeStruct((N_EDGES, D), jnp.float32),
            jax.ShapeDtypeStruct((N_EDGES,), jnp.float32),
        ],
    )(g1s, g2d, nea, mg)


def _final_body(ug_ref, vg_ref, ea_ref, wf1e_ref, dv_ref, tv_ref, wrow_d_ref,
                wrow_t_ref, bf1_ref, wf2_ref, out_ref):
    eh = (ug_ref[...] + vg_ref[...] + ea_ref[...] @ wf1e_ref[...]
          + dv_ref[...][:, None] * wrow_d_ref[...]
          + tv_ref[...][:, None] * wrow_t_ref[...] + bf1_ref[...])
    eh = jnp.maximum(eh, 0.0)
    logits = eh @ wf2_ref[...]
    out_ref[...] = jax.nn.sigmoid(logits[:, 0])


def _final_stage(ug, vg, ea, wf1e, dv, tv, wrow_d, wrow_t, bf1, wf2):
    grid = (N_EDGES // EBLK,)
    ein = ea.shape[1]
    return pl.pallas_call(
        _final_body,
        grid=grid,
        in_specs=[
            pl.BlockSpec((EBLK, D), lambda i: (i, 0)),
            pl.BlockSpec((EBLK, D), lambda i: (i, 0)),
            pl.BlockSpec((EBLK, ein), lambda i: (i, 0)),
            pl.BlockSpec((ein, D), lambda i: (0, 0)),
            pl.BlockSpec((EBLK,), lambda i: (i,)),
            pl.BlockSpec((EBLK,), lambda i: (i,)),
            pl.BlockSpec((1, D), lambda i: (0, 0)),
            pl.BlockSpec((1, D), lambda i: (0, 0)),
            pl.BlockSpec((1, D), lambda i: (0, 0)),
            pl.BlockSpec((D, 1), lambda i: (0, 0)),
        ],
        out_specs=pl.BlockSpec((EBLK,), lambda i: (i,)),
        out_shape=jax.ShapeDtypeStruct((N_EDGES,), jnp.float32),
    )(ug, vg, ea, wf1e, dv, tv, wrow_d[None, :], wrow_t[None, :], bf1[None, :], wf2)


def kernel(node_attr, pos, edge_attr, spec, Wp, bp, g0, b0, We1, be1, We2, be2, att, Wm, bm, Wv, bv, Ws, bs, gam, bet, Wf1, bf1, Wf2, bf2, edge_index, batch, source, sink):
    src, dst = edge_index[0], edge_index[1]
    x = jax.nn.relu(_bn(node_attr @ Wp + bp, g0, b0))
    ea = edge_attr
    node_values = None
    for l in range(L_LAYERS):
        We1l, be1l, We2l, be2l = We1[l], be1[l], We2[l], be2[l]
        attl, Wml, bml = att[l], Wm[l], bm[l]
        A = x @ We1l[0:128] - pos @ We1l[260:262] + spec @ We1l[262:264]
        B = x @ We1l[128:256] + pos @ We1l[260:262]
        a1, a2, ae = attl[0:128], attl[128:256], attl[256:260]
        ne, nea = _encoder(A[src], B[dst], ea, We1l[256:260], be1l, We2l,
                           be2l, ae)
        xk = x
        for _ in range(K_HOPS):
            g1 = xk @ a1
            g2 = xk @ a2
            m = xk @ Wml + bml
            msg, w = _hop_stage(g1[src], g2[dst], nea, m[src])
            s = jax.ops.segment_sum(w, dst, num_segments=N_NODES)
            rowacc = jax.ops.segment_sum(msg, dst, num_segments=N_NODES)
            xk = jax.nn.relu(rowacc / (s[:, None] + 1e-16)) + xk
        node_values = (xk @ Wv[l] + bv[l])[:, 0]
        ea = ne
        x = jax.nn.relu(_bn(xk, gam[l], bet[l]) + x)
    u = x @ Wf1[0:128]
    v = x @ Wf1[132:260]
    fv, tv = node_values[src], node_values[dst]
    return _final_stage(u[src], v[dst], ea, Wf1[128:132], tv - fv, tv,
                        Wf1[260], Wf1[261], bf1, Wf2), node_values


# speedup vs baseline: 4.0607x; 3.9528x over previous
import jax
import jax.numpy as jnp
from jax import lax
from jax.experimental import pallas as pl
from jax.experimental.pallas import tpu as pltpu
from jax.experimental.pallas import tpu_sc as plsc

N_NODES = 10000
N_EDGES = 320000
D = 128
L_LAYERS = 2
K_HOPS = 2

EBLK = 512

NCORE = 2
NSUB = 16
NW = NCORE * NSUB
EPW = N_EDGES // NW
CHK = 80
NCHK = EPW // CHK
AW = 144
NPAD = 10240
RPS = NPAD // NSUB

_SC_PARAMS = pltpu.CompilerParams(use_tc_tiling_on_sc=False,
                                  needs_layout_passes=False)


def _lane_bcast(vec16, lane_idx16):
    return lax.gather(
        vec16, lane_idx16[:, None],
        lax.GatherDimensionNumbers(offset_dims=(), collapsed_slice_dims=(0,),
                                   start_index_map=(0,)),
        slice_sizes=(1,), mode=lax.GatherScatterMode.PROMISE_IN_BOUNDS)


def _mm_body(x_ref, w_ref, b_ref, out_ref):
    out_ref[...] = x_ref[...] @ w_ref[...] + b_ref[...]


def _tc_matmul(x, w, b, nblk):
    n, kdim = x.shape
    mdim = w.shape[1]
    return pl.pallas_call(
        _mm_body,
        grid=(n // nblk,),
        in_specs=[
            pl.BlockSpec((nblk, kdim), lambda i: (i, 0)),
            pl.BlockSpec((kdim, mdim), lambda i: (0, 0)),
            pl.BlockSpec((1, mdim), lambda i: (0, 0)),
        ],
        out_specs=pl.BlockSpec((nblk, mdim), lambda i: (i, 0)),
        out_shape=jax.ShapeDtypeStruct((n, mdim), jnp.float32),
    )(x, w, b[None])


def _nepost_body(eh_ref, we2_ref, be2_ref, ae_ref, ne_ref, nea_ref):
    ne = eh_ref[...] @ we2_ref[...] + be2_ref[...]
    ne_ref[...] = ne
    nea_ref[...] = ne @ ae_ref[...]


def _tc_nepost(eh, we2, be2, ae):
    return pl.pallas_call(
        _nepost_body,
        grid=(N_EDGES // EBLK,),
        in_specs=[
            pl.BlockSpec((EBLK, D), lambda i: (i, 0)),
            pl.BlockSpec((D, 4), lambda i: (0, 0)),
            pl.BlockSpec((1, 4), lambda i: (0, 0)),
            pl.BlockSpec((4,), lambda i: (0,)),
        ],
        out_specs=[
            pl.BlockSpec((EBLK, 4), lambda i: (i, 0)),
            pl.BlockSpec((EBLK,), lambda i: (i,)),
        ],
        out_shape=[
            jax.ShapeDtypeStruct((N_EDGES, 4), jnp.float32),
            jax.ShapeDtypeStruct((N_EDGES,), jnp.float32),
        ],
    )(eh, we2, be2[None], ae)


def _finpost_body(ehf_ref, wf2_ref, bf2_ref, out_ref):
    logits = (ehf_ref[...] @ wf2_ref[...])[:, 0] + bf2_ref[0, 0]
    out_ref[...] = jax.nn.sigmoid(logits)


def _tc_finpost(ehf, wf2, bf2):
    return pl.pallas_call(
        _finpost_body,
        grid=(N_EDGES // EBLK,),
        in_specs=[
            pl.BlockSpec((EBLK, D), lambda i: (i, 0)),
            pl.BlockSpec((D, 1), lambda i: (0, 0)),
            pl.BlockSpec((1, 1), lambda i: (0, 0)),
        ],
        out_specs=pl.BlockSpec((EBLK,), lambda i: (i,)),
        out_shape=jax.ShapeDtypeStruct((N_EDGES,), jnp.float32),
    )(ehf, wf2, bf2[None])


def _sc_enc_body(a_hbm, b_hbm, src_hbm, dst_hbm, eaw_hbm, eh_hbm,
                 idx_s_v, idx_d_v, arows_v, brows_v, erows_v, eh_v, sem):
    c = lax.axis_index("c")
    s = lax.axis_index("s")
    base = (c * NSUB + s) * EPW

    def chunk(i, carry):
        off = base + i * CHK
        pltpu.sync_copy(src_hbm.at[pl.ds(off, CHK)], idx_s_v)
        pltpu.sync_copy(dst_hbm.at[pl.ds(off, CHK)], idx_d_v)
        pltpu.sync_copy(eaw_hbm.at[pl.ds(off, CHK)], erows_v)
        pltpu.async_copy(a_hbm.at[idx_s_v], arows_v, sem).wait()
        pltpu.async_copy(b_hbm.at[idx_d_v], brows_v, sem).wait()
        for e in range(CHK):
            for j in range(D // 16):
                sl = pl.ds(j * 16, 16)
                eh_v[e, sl] = jnp.maximum(
                    arows_v[e, sl] + brows_v[e, sl] + erows_v[e, sl], 0.0)
        pltpu.sync_copy(eh_v, eh_hbm.at[pl.ds(off, CHK)])
        return carry

    lax.fori_loop(0, NCHK, chunk, 0)


def _sc_enc(A, B, src, dst, eaw):
    mesh = plsc.VectorSubcoreMesh(core_axis_name="c", subcore_axis_name="s")
    f = pl.kernel(
        _sc_enc_body, mesh=mesh, compiler_params=_SC_PARAMS,
        out_type=jax.ShapeDtypeStruct((N_EDGES, D), jnp.float32),
        scratch_types=[
            pltpu.VMEM((CHK,), jnp.int32),
            pltpu.VMEM((CHK,), jnp.int32),
            pltpu.VMEM((CHK, D), jnp.float32),
            pltpu.VMEM((CHK, D), jnp.float32),
            pltpu.VMEM((CHK, D), jnp.float32),
            pltpu.VMEM((CHK, D), jnp.float32),
            pltpu.SemaphoreType.DMA,
        ],
    )
    return f(A, B, src, dst, eaw)


def _sc_hop_body(mext_hbm, g2e_hbm, src_hbm, dst_hbm, nea_hbm, zeros_hbm,
                 out_hbm, idx_s_v, idx_d_v, rows_v, msg_v, nea_v, g2r_v,
                 acc, sem):
    c = lax.axis_index("c")
    s = lax.axis_index("s")
    base = (c * NSUB + s) * EPW
    pltpu.sync_copy(zeros_hbm.at[pl.ds(s * RPS, RPS)],
                    acc.at[pl.ds(s * RPS, RPS)])
    plsc.subcore_barrier()

    iota16 = lax.iota(jnp.int32, 16)
    onehot0 = jnp.where(iota16 == 0, 1.0, 0.0)

    def chunk(i, carry):
        off = base + i * CHK
        pltpu.sync_copy(src_hbm.at[pl.ds(off, CHK)], idx_s_v)
        pltpu.sync_copy(dst_hbm.at[pl.ds(off, CHK)], idx_d_v)
        pltpu.sync_copy(nea_hbm.at[pl.ds(off, CHK)], nea_v)
        pltpu.async_copy(mext_hbm.at[idx_s_v], rows_v, sem).wait()
        pltpu.async_copy(g2e_hbm.at[idx_d_v], g2r_v, sem).wait()
        for k in range(CHK // 16):
            nea16 = nea_v[pl.ds(k * 16, 16)]
            for t in range(16):
                e = k * 16 + t
                gsum = rows_v[e, pl.ds(D, 16)] + g2r_v[e, :]
                lgv = gsum + _lane_bcast(nea16, jnp.full((16,), t, jnp.int32))
                lgv = jnp.where(lgv >= 0.0, lgv, 0.2 * lgv)
                wv = jnp.exp(lgv)
                wb = _lane_bcast(wv, jnp.zeros((16,), jnp.int32))
                for j in range(D // 16):
                    msg_v[e, pl.ds(j * 16, 16)] = (
                        rows_v[e, pl.ds(j * 16, 16)] * wb)
                msg_v[e, pl.ds(D, 16)] = onehot0 * wb
        pltpu.sync_copy(msg_v, acc.at[idx_d_v], add=True)
        return carry

    lax.fori_loop(0, NCHK, chunk, 0)
    plsc.subcore_barrier()
    pltpu.sync_copy(acc.at[pl.ds(s * RPS, RPS)],
                    out_hbm.at[c, pl.ds(s * RPS, RPS)])


def _sc_hop(mext, g2e, src, dst, nea):
    mesh = plsc.VectorSubcoreMesh(core_axis_name="c", subcore_axis_name="s")
    f = pl.kernel(
        _sc_hop_body, mesh=mesh, compiler_params=_SC_PARAMS,
        out_type=jax.ShapeDtypeStruct((NCORE, NPAD, AW), jnp.float32),
        scratch_types=[
            pltpu.VMEM((CHK,), jnp.int32),
            pltpu.VMEM((CHK,), jnp.int32),
            pltpu.VMEM((CHK, AW), jnp.float32),
            pltpu.VMEM((CHK, AW), jnp.float32),
            pltpu.VMEM((CHK,), jnp.float32),
            pltpu.VMEM((CHK, 16), jnp.float32),
            pltpu.VMEM_SHARED((NPAD, AW), jnp.float32),
            pltpu.SemaphoreType.DMA,
        ],
    )
    zeros = jnp.zeros((NPAD, AW), jnp.float32)
    return f(mext, g2e, src, dst, nea, zeros)


def _sc_fin_body(u_hbm, v_hbm, src_hbm, dst_hbm, eafw_hbm, wdt_hbm,
                 ehf_hbm, idx_s_v, idx_d_v, urows_v, vrows_v, erows_v,
                 ehf_v, wdt_v, sem):
    c = lax.axis_index("c")
    s = lax.axis_index("s")
    base = (c * NSUB + s) * EPW
    pltpu.sync_copy(wdt_hbm, wdt_v)
    lane0 = jnp.zeros((16,), jnp.int32)

    def chunk(i, carry):
        off = base + i * CHK
        pltpu.sync_copy(src_hbm.at[pl.ds(off, CHK)], idx_s_v)
        pltpu.sync_copy(dst_hbm.at[pl.ds(off, CHK)], idx_d_v)
        pltpu.sync_copy(eafw_hbm.at[pl.ds(off, CHK)], erows_v)
        pltpu.async_copy(u_hbm.at[idx_s_v], urows_v, sem).wait()
        pltpu.async_copy(v_hbm.at[idx_d_v], vrows_v, sem).wait()
        for e in range(CHK):
            fv = _lane_bcast(urows_v[e, pl.ds(D, 16)], lane0)
            tv = _lane_bcast(vrows_v[e, pl.ds(D, 16)], lane0)
            dv = tv - fv
            for j in range(D // 16):
                sl = pl.ds(j * 16, 16)
                ehf_v[e, sl] = jnp.maximum(
                    urows_v[e, sl] + vrows_v[e, sl] + erows_v[e, sl]
                    + dv * wdt_v[0, sl] + tv * wdt_v[1, sl], 0.0)
        pltpu.sync_copy(ehf_v, ehf_hbm.at[pl.ds(off, CHK)])
        return carry

    lax.fori_loop(0, NCHK, chunk, 0)


def _sc_fin(U, V, src, dst, eafw, wdt):
    mesh = plsc.VectorSubcoreMesh(core_axis_name="c", subcore_axis_name="s")
    f = pl.kernel(
        _sc_fin_body, mesh=mesh, compiler_params=_SC_PARAMS,
        out_type=jax.ShapeDtypeStruct((N_EDGES, D), jnp.float32),
        scratch_types=[
            pltpu.VMEM((CHK,), jnp.int32),
            pltpu.VMEM((CHK,), jnp.int32),
            pltpu.VMEM((CHK, AW), jnp.float32),
            pltpu.VMEM((CHK, AW), jnp.float32),
            pltpu.VMEM((CHK, D), jnp.float32),
            pltpu.VMEM((CHK, D), jnp.float32),
            pltpu.VMEM((2, D), jnp.float32),
            pltpu.SemaphoreType.DMA,
        ],
    )
    return f(U, V, src, dst, eafw, wdt)


def _bn(x, gamma, beta, eps=1e-5):
    mu = jnp.mean(x, axis=0, keepdims=True)
    var = jnp.var(x, axis=0, keepdims=True)
    return gamma * (x - mu) / jnp.sqrt(var + eps) + beta


def kernel(node_attr, pos, edge_attr, spec, Wp, bp, g0, b0, We1, be1, We2, be2, att, Wm, bm, Wv, bv, Ws, bs, gam, bet, Wf1, bf1, Wf2, bf2, edge_index, batch, source, sink):
    src, dst = edge_index[0], edge_index[1]
    x = jax.nn.relu(_bn(_tc_matmul(node_attr, Wp, bp, 1000), g0, b0))
    ea = edge_attr
    zerosD = jnp.zeros((D,), jnp.float32)
    pad15 = jnp.zeros((N_NODES, 15), jnp.float32)
    node_values = None
    for l in range(L_LAYERS):
        We1l, be1l, We2l, be2l = We1[l], be1[l], We2[l], be2[l]
        attl, Wml, bml = att[l], Wm[l], bm[l]
        a1, a2, ae = attl[0:128], attl[128:256], attl[256:260]
        nodecat = jnp.concatenate([x, pos, spec], axis=1)
        wa = jnp.concatenate([We1l[0:128], -We1l[260:262], We1l[262:264]], 0)
        wb = jnp.concatenate([We1l[128:256], We1l[260:262],
                              jnp.zeros((2, D), jnp.float32)], 0)
        A = _tc_matmul(nodecat, wa, zerosD, 1000)
        B = _tc_matmul(nodecat, wb, zerosD, 1000)
        eaw = _tc_matmul(ea, We1l[256:260], be1l, EBLK)
        eh = _sc_enc(A, B, src, dst, eaw)
        ne, nea = _tc_nepost(eh, We2l, be2l, ae)
        xk = x
        for _ in range(K_HOPS):
            wmg = jnp.concatenate([Wml, a1[:, None], a2[:, None]], axis=1)
            bmg = jnp.concatenate([bml, jnp.zeros((2,), jnp.float32)])
            mg = _tc_matmul(xk, wmg, bmg, 1000)
            mext = jnp.concatenate([mg[:, 0:129], pad15], axis=1)
            g2e = jnp.concatenate([mg[:, 129:130],
                                   jnp.zeros((N_NODES, 15), jnp.float32)], 1)
            parts = _sc_hop(mext, g2e, src, dst, nea)
            tot = parts[0, :N_NODES] + parts[1, :N_NODES]
            xk = jax.nn.relu(tot[:, :D] / (tot[:, D:D + 1] + 1e-16)) + xk
        node_values = (xk @ Wv[l] + bv[l])[:, 0]
        ea = ne
        x = jax.nn.relu(_bn(xk, gam[l], bet[l]) + x)
    wuv = jnp.concatenate([Wf1[0:128], Wf1[132:260]], axis=1)
    uv = _tc_matmul(x, wuv, jnp.zeros((2 * D,), jnp.float32), 1000)
    U = jnp.concatenate([uv[:, 0:D], node_values[:, None], pad15], axis=1)
    V = jnp.concatenate([uv[:, D:2 * D], node_values[:, None], pad15], axis=1)
    eafw = _tc_matmul(ea, Wf1[128:132], bf1, EBLK)
    wdt = jnp.stack([Wf1[260], Wf1[261]], axis=0)
    ehf = _sc_fin(U, V, src, dst, eafw, wdt)
    return _tc_finpost(ehf, Wf2, bf2), node_values


# speedup vs baseline: 6.0528x; 1.4906x over previous
import jax
import jax.numpy as jnp
from jax import lax
from jax.experimental import pallas as pl
from jax.experimental.pallas import tpu as pltpu
from jax.experimental.pallas import tpu_sc as plsc

N_NODES = 10000
N_EDGES = 320000
D = 128
L_LAYERS = 2
K_HOPS = 2

EBLK = 512

NCORE = 2
NSUB = 16
NW = NCORE * NSUB
EPW = N_EDGES // NW
CHK = 80
NCHK = EPW // CHK
AW = 144
NPAD = 10240
RPS = NPAD // NSUB

_SC_PARAMS = pltpu.CompilerParams(use_tc_tiling_on_sc=False,
                                  needs_layout_passes=False)


def _lane_bcast(vec16, lane_idx16):
    return lax.gather(
        vec16, lane_idx16[:, None],
        lax.GatherDimensionNumbers(offset_dims=(), collapsed_slice_dims=(0,),
                                   start_index_map=(0,)),
        slice_sizes=(1,), mode=lax.GatherScatterMode.PROMISE_IN_BOUNDS)


def _mm_body(x_ref, w_ref, b_ref, out_ref):
    out_ref[...] = x_ref[...] @ w_ref[...] + b_ref[...]


def _tc_matmul(x, w, b, nblk):
    n, kdim = x.shape
    mdim = w.shape[1]
    return pl.pallas_call(
        _mm_body,
        grid=(n // nblk,),
        in_specs=[
            pl.BlockSpec((nblk, kdim), lambda i: (i, 0)),
            pl.BlockSpec((kdim, mdim), lambda i: (0, 0)),
            pl.BlockSpec((1, mdim), lambda i: (0, 0)),
        ],
        out_specs=pl.BlockSpec((nblk, mdim), lambda i: (i, 0)),
        out_shape=jax.ShapeDtypeStruct((n, mdim), jnp.float32),
    )(x, w, b[None])


def _nepost_body(eh_ref, we2_ref, be2_ref, ae_ref, ne_ref, nea_ref):
    ne = eh_ref[...] @ we2_ref[...] + be2_ref[...]
    ne_ref[...] = ne
    nea_ref[...] = ne @ ae_ref[...]


def _tc_nepost(eh, we2, be2, ae):
    return pl.pallas_call(
        _nepost_body,
        grid=(N_EDGES // EBLK,),
        in_specs=[
            pl.BlockSpec((EBLK, D), lambda i: (i, 0)),
            pl.BlockSpec((D, 4), lambda i: (0, 0)),
            pl.BlockSpec((1, 4), lambda i: (0, 0)),
            pl.BlockSpec((4,), lambda i: (0,)),
        ],
        out_specs=[
            pl.BlockSpec((EBLK, 4), lambda i: (i, 0)),
            pl.BlockSpec((EBLK,), lambda i: (i,)),
        ],
        out_shape=[
            jax.ShapeDtypeStruct((N_EDGES, 4), jnp.float32),
            jax.ShapeDtypeStruct((N_EDGES,), jnp.float32),
        ],
    )(eh, we2, be2[None], ae)


def _finpost_body(ehf_ref, wf2_ref, bf2_ref, out_ref):
    logits = (ehf_ref[...] @ wf2_ref[...])[:, 0] + bf2_ref[0, 0]
    out_ref[...] = jax.nn.sigmoid(logits)


def _tc_finpost(ehf, wf2, bf2):
    return pl.pallas_call(
        _finpost_body,
        grid=(N_EDGES // EBLK,),
        in_specs=[
            pl.BlockSpec((EBLK, D), lambda i: (i, 0)),
            pl.BlockSpec((D, 1), lambda i: (0, 0)),
            pl.BlockSpec((1, 1), lambda i: (0, 0)),
        ],
        out_specs=pl.BlockSpec((EBLK,), lambda i: (i,)),
        out_shape=jax.ShapeDtypeStruct((N_EDGES,), jnp.float32),
    )(ehf, wf2, bf2[None])


def _sc_enc_body(a_hbm, b_hbm, src_hbm, dst_hbm, eaw_hbm, eh_hbm,
                 idx_s_v, idx_d_v, arows_v, brows_v, erows_v, eh_v, sems):
    c = lax.axis_index("c")
    s = lax.axis_index("s")
    base = (c * NSUB + s) * EPW

    def prefetch(i, b):
        off = base + i * CHK
        pltpu.sync_copy(src_hbm.at[pl.ds(off, CHK)], idx_s_v.at[b])
        pltpu.sync_copy(dst_hbm.at[pl.ds(off, CHK)], idx_d_v.at[b])
        pltpu.async_copy(eaw_hbm.at[pl.ds(off, CHK)], erows_v.at[b], sems[b])
        pltpu.async_copy(a_hbm.at[idx_s_v.at[b]], arows_v.at[b], sems[b])
        pltpu.async_copy(b_hbm.at[idx_d_v.at[b]], brows_v.at[b], sems[b])

    for b in range(2):
        prefetch(b, b)

    def pair(g, carry):
        for b in range(2):
            i = 2 * g + b

            @pl.when(i < NCHK)
            def _process():
                off = base + i * CHK
                pltpu.make_async_copy(eaw_hbm.at[pl.ds(off, CHK)],
                                      erows_v.at[b], sems[b]).wait()
                pltpu.make_async_copy(a_hbm.at[idx_s_v.at[b]],
                                      arows_v.at[b], sems[b]).wait()
                pltpu.make_async_copy(b_hbm.at[idx_d_v.at[b]],
                                      brows_v.at[b], sems[b]).wait()

                def edge(e, cc):
                    for j in range(D // 16):
                        sl = pl.ds(j * 16, 16)
                        eh_v[b, e, sl] = jnp.maximum(
                            arows_v[b, e, sl] + brows_v[b, e, sl]
                            + erows_v[b, e, sl], 0.0)
                    return cc

                lax.fori_loop(0, CHK, edge, 0)
                pltpu.sync_copy(eh_v.at[b], eh_hbm.at[pl.ds(off, CHK)])

                @pl.when(i + 2 < NCHK)
                def _():
                    prefetch(i + 2, b)
        return carry

    lax.fori_loop(0, (NCHK + 1) // 2, pair, 0)


def _sc_enc(A, B, src, dst, eaw):
    mesh = plsc.VectorSubcoreMesh(core_axis_name="c", subcore_axis_name="s")
    f = pl.kernel(
        _sc_enc_body, mesh=mesh, compiler_params=_SC_PARAMS,
        out_type=jax.ShapeDtypeStruct((N_EDGES, D), jnp.float32),
        scratch_types=[
            pltpu.VMEM((2, CHK), jnp.int32),
            pltpu.VMEM((2, CHK), jnp.int32),
            pltpu.VMEM((2, CHK, D), jnp.float32),
            pltpu.VMEM((2, CHK, D), jnp.float32),
            pltpu.VMEM((2, CHK, D), jnp.float32),
            pltpu.VMEM((2, CHK, D), jnp.float32),
            [pltpu.SemaphoreType.DMA, pltpu.SemaphoreType.DMA],
        ],
    )
    return f(A, B, src, dst, eaw)


def _sc_hop_body(mext_hbm, g2e_hbm, src_hbm, dst_hbm, nea_hbm, zeros_hbm,
                 out_hbm, idx_s_v, idx_d_v, rows_v, msg_v, nea_v, g2r_v,
                 acc, sem):
    c = lax.axis_index("c")
    s = lax.axis_index("s")
    base = (c * NSUB + s) * EPW
    pltpu.sync_copy(zeros_hbm.at[pl.ds(s * RPS, RPS)],
                    acc.at[pl.ds(s * RPS, RPS)])
    plsc.subcore_barrier()

    iota16 = lax.iota(jnp.int32, 16)
    onehot0 = jnp.where(iota16 == 0, 1.0, 0.0)

    def chunk(i, carry):
        off = base + i * CHK
        pltpu.sync_copy(src_hbm.at[pl.ds(off, CHK)], idx_s_v)
        pltpu.sync_copy(dst_hbm.at[pl.ds(off, CHK)], idx_d_v)
        pltpu.sync_copy(nea_hbm.at[pl.ds(off, CHK)], nea_v)
        pltpu.async_copy(mext_hbm.at[idx_s_v], rows_v, sem).wait()
        pltpu.async_copy(g2e_hbm.at[idx_d_v], g2r_v, sem).wait()
        for k in range(CHK // 16):
            nea16 = nea_v[pl.ds(k * 16, 16)]
            for t in range(16):
                e = k * 16 + t
                gsum = rows_v[e, pl.ds(D, 16)] + g2r_v[e, :]
                lgv = gsum + _lane_bcast(nea16, jnp.full((16,), t, jnp.int32))
                lgv = jnp.where(lgv >= 0.0, lgv, 0.2 * lgv)
                wv = jnp.exp(lgv)
                wb = _lane_bcast(wv, jnp.zeros((16,), jnp.int32))
                for j in range(D // 16):
                    msg_v[e, pl.ds(j * 16, 16)] = (
                        rows_v[e, pl.ds(j * 16, 16)] * wb)
                msg_v[e, pl.ds(D, 16)] = onehot0 * wb
        pltpu.sync_copy(msg_v, acc.at[idx_d_v], add=True)
        return carry

    lax.fori_loop(0, NCHK, chunk, 0)
    plsc.subcore_barrier()
    pltpu.sync_copy(acc.at[pl.ds(s * RPS, RPS)],
                    out_hbm.at[c, pl.ds(s * RPS, RPS)])


def _sc_hop(mext, g2e, src, dst, nea):
    mesh = plsc.VectorSubcoreMesh(core_axis_name="c", subcore_axis_name="s")
    f = pl.kernel(
        _sc_hop_body, mesh=mesh, compiler_params=_SC_PARAMS,
        out_type=jax.ShapeDtypeStruct((NCORE, NPAD, AW), jnp.float32),
        scratch_types=[
            pltpu.VMEM((CHK,), jnp.int32),
            pltpu.VMEM((CHK,), jnp.int32),
            pltpu.VMEM((CHK, AW), jnp.float32),
            pltpu.VMEM((CHK, AW), jnp.float32),
            pltpu.VMEM((CHK,), jnp.float32),
            pltpu.VMEM((CHK, 16), jnp.float32),
            pltpu.VMEM_SHARED((NPAD, AW), jnp.float32),
            pltpu.SemaphoreType.DMA,
        ],
    )
    zeros = jnp.zeros((NPAD, AW), jnp.float32)
    return f(mext, g2e, src, dst, nea, zeros)


def _sc_fin_body(u_hbm, v_hbm, src_hbm, dst_hbm, eafw_hbm, wdt_hbm,
                 ehf_hbm, idx_s_v, idx_d_v, urows_v, vrows_v, erows_v,
                 ehf_v, wdt_v, sems):
    c = lax.axis_index("c")
    s = lax.axis_index("s")
    base = (c * NSUB + s) * EPW
    pltpu.sync_copy(wdt_hbm, wdt_v)
    lane0 = jnp.zeros((16,), jnp.int32)

    def prefetch(i, b):
        off = base + i * CHK
        pltpu.sync_copy(src_hbm.at[pl.ds(off, CHK)], idx_s_v.at[b])
        pltpu.sync_copy(dst_hbm.at[pl.ds(off, CHK)], idx_d_v.at[b])
        pltpu.async_copy(eafw_hbm.at[pl.ds(off, CHK)], erows_v.at[b], sems[b])
        pltpu.async_copy(u_hbm.at[idx_s_v.at[b]], urows_v.at[b], sems[b])
        pltpu.async_copy(v_hbm.at[idx_d_v.at[b]], vrows_v.at[b], sems[b])

    for b in range(2):
        prefetch(b, b)

    def pair(g, carry):
        for b in range(2):
            i = 2 * g + b

            @pl.when(i < NCHK)
            def _process():
                off = base + i * CHK
                pltpu.make_async_copy(eafw_hbm.at[pl.ds(off, CHK)],
                                      erows_v.at[b], sems[b]).wait()
                pltpu.make_async_copy(u_hbm.at[idx_s_v.at[b]],
                                      urows_v.at[b], sems[b]).wait()
                pltpu.make_async_copy(v_hbm.at[idx_d_v.at[b]],
                                      vrows_v.at[b], sems[b]).wait()

                def edge(e, cc):
                    fv = _lane_bcast(urows_v[b, e, pl.ds(D, 16)], lane0)
                    tv = _lane_bcast(vrows_v[b, e, pl.ds(D, 16)], lane0)
                    dv = tv - fv
                    for j in range(D // 16):
                        sl = pl.ds(j * 16, 16)
                        ehf_v[b, e, sl] = jnp.maximum(
                            urows_v[b, e, sl] + vrows_v[b, e, sl]
                            + erows_v[b, e, sl]
                            + dv * wdt_v[0, sl] + tv * wdt_v[1, sl], 0.0)
                    return cc

                lax.fori_loop(0, CHK, edge, 0)
                pltpu.sync_copy(ehf_v.at[b], ehf_hbm.at[pl.ds(off, CHK)])

                @pl.when(i + 2 < NCHK)
                def _():
                    prefetch(i + 2, b)
        return carry

    lax.fori_loop(0, (NCHK + 1) // 2, pair, 0)


def _sc_fin(U, V, src, dst, eafw, wdt):
    mesh = plsc.VectorSubcoreMesh(core_axis_name="c", subcore_axis_name="s")
    f = pl.kernel(
        _sc_fin_body, mesh=mesh, compiler_params=_SC_PARAMS,
        out_type=jax.ShapeDtypeStruct((N_EDGES, D), jnp.float32),
        scratch_types=[
            pltpu.VMEM((2, CHK), jnp.int32),
            pltpu.VMEM((2, CHK), jnp.int32),
            pltpu.VMEM((2, CHK, AW), jnp.float32),
            pltpu.VMEM((2, CHK, AW), jnp.float32),
            pltpu.VMEM((2, CHK, D), jnp.float32),
            pltpu.VMEM((2, CHK, D), jnp.float32),
            pltpu.VMEM((2, D), jnp.float32),
            [pltpu.SemaphoreType.DMA, pltpu.SemaphoreType.DMA],
        ],
    )
    return f(U, V, src, dst, eafw, wdt)


def _bn(x, gamma, beta, eps=1e-5):
    mu = jnp.mean(x, axis=0, keepdims=True)
    var = jnp.var(x, axis=0, keepdims=True)
    return gamma * (x - mu) / jnp.sqrt(var + eps) + beta


def kernel(node_attr, pos, edge_attr, spec, Wp, bp, g0, b0, We1, be1, We2, be2, att, Wm, bm, Wv, bv, Ws, bs, gam, bet, Wf1, bf1, Wf2, bf2, edge_index, batch, source, sink):
    src, dst = edge_index[0], edge_index[1]
    x = jax.nn.relu(_bn(_tc_matmul(node_attr, Wp, bp, 1000), g0, b0))
    ea = edge_attr
    zerosD = jnp.zeros((D,), jnp.float32)
    pad15 = jnp.zeros((N_NODES, 15), jnp.float32)
    node_values = None
    for l in range(L_LAYERS):
        We1l, be1l, We2l, be2l = We1[l], be1[l], We2[l], be2[l]
        attl, Wml, bml = att[l], Wm[l], bm[l]
        a1, a2, ae = attl[0:128], attl[128:256], attl[256:260]
        nodecat = jnp.concatenate([x, pos, spec], axis=1)
        wa = jnp.concatenate([We1l[0:128], -We1l[260:262], We1l[262:264]], 0)
        wb = jnp.concatenate([We1l[128:256], We1l[260:262],
                              jnp.zeros((2, D), jnp.float32)], 0)
        A = _tc_matmul(nodecat, wa, zerosD, 1000)
        B = _tc_matmul(nodecat, wb, zerosD, 1000)
        eaw = _tc_matmul(ea, We1l[256:260], be1l, EBLK)
        eh = _sc_enc(A, B, src, dst, eaw)
        ne, nea = _tc_nepost(eh, We2l, be2l, ae)
        xk = x
        for _ in range(K_HOPS):
            wmg = jnp.concatenate([Wml, a1[:, None], a2[:, None]], axis=1)
            bmg = jnp.concatenate([bml, jnp.zeros((2,), jnp.float32)])
            mg = _tc_matmul(xk, wmg, bmg, 1000)
            mext = jnp.concatenate([mg[:, 0:129], pad15], axis=1)
            g2e = jnp.concatenate([mg[:, 129:130],
                                   jnp.zeros((N_NODES, 15), jnp.float32)], 1)
            parts = _sc_hop(mext, g2e, src, dst, nea)
            tot = parts[0, :N_NODES] + parts[1, :N_NODES]
            xk = jax.nn.relu(tot[:, :D] / (tot[:, D:D + 1] + 1e-16)) + xk
        node_values = (xk @ Wv[l] + bv[l])[:, 0]
        ea = ne
        x = jax.nn.relu(_bn(xk, gam[l], bet[l]) + x)
    wuv = jnp.concatenate([Wf1[0:128], Wf1[132:260]], axis=1)
    uv = _tc_matmul(x, wuv, jnp.zeros((2 * D,), jnp.float32), 1000)
    U = jnp.concatenate([uv[:, 0:D], node_values[:, None], pad15], axis=1)
    V = jnp.concatenate([uv[:, D:2 * D], node_values[:, None], pad15], axis=1)
    eafw = _tc_matmul(ea, Wf1[128:132], bf1, EBLK)
    wdt = jnp.stack([Wf1[260], Wf1[261]], axis=0)
    ehf = _sc_fin(U, V, src, dst, eafw, wdt)
    return _tc_finpost(ehf, Wf2, bf2), node_values
